# baseline (speedup 1.0000x reference)
import functools

import jax
import jax.numpy as jnp
from jax import lax
from jax.experimental import pallas as pl
from jax.experimental.pallas import tpu as pltpu
from jax.experimental.pallas import tpu_sc as plsc

_N = 10000
_NFEAT = 128
_NHID = 32
_HPAD = 128
_B = 16384

_BM = 200
_NBLK = _N // _BM


def _gcn_tc_body(x_ref, w1_ref, b1_ref, w2_ref, b2_ref, adj_ref, h_ref,
                 s1_scr, s2_scr):
    p = pl.program_id(0)
    i = pl.program_id(1)

    @pl.when((p == 0) & (i == 0))
    def _():
        s1_scr[...] = jnp.dot(x_ref[...], w1_ref[...],
                              preferred_element_type=jnp.float32)

    @pl.when(p == 0)
    def _():
        h1 = jnp.dot(adj_ref[...], s1_scr[...],
                     preferred_element_type=jnp.float32) + b1_ref[...]
        h1 = jnp.maximum(h1, 0.0)
        s2_scr[pl.ds(i * _BM, _BM), :] = jnp.dot(
            h1, w2_ref[...], preferred_element_type=jnp.float32)

    @pl.when(p == 1)
    def _():
        blk = _NBLK - 1 - i
        h_ref[pl.ds(blk * _BM, _BM), :] = jnp.dot(
            adj_ref[...], s2_scr[...],
            preferred_element_type=jnp.float32) + b2_ref[...]


def _gcn_tc(x, adj, W1, b1, W2p, b2p):
    return pl.pallas_call(
        _gcn_tc_body,
        grid=(2, _NBLK),
        in_specs=[
            pl.BlockSpec((_N, _NFEAT), lambda p, i: (0, 0)),
            pl.BlockSpec((_NFEAT, _NHID), lambda p, i: (0, 0)),
            pl.BlockSpec((1, _NHID), lambda p, i: (0, 0)),
            pl.BlockSpec((_NHID, _HPAD), lambda p, i: (0, 0)),
            pl.BlockSpec((1, _HPAD), lambda p, i: (0, 0)),
            pl.BlockSpec((_BM, _N),
                         lambda p, i: (jnp.where(p == 1, _NBLK - 1 - i, i),
                                       0)),
        ],
        out_specs=pl.BlockSpec((_N, _HPAD), lambda p, i: (0, 0)),
        out_shape=jax.ShapeDtypeStruct((_N, _HPAD), jnp.float32),
        scratch_shapes=[
            pltpu.VMEM((_N, _NHID), jnp.float32),
            pltpu.VMEM((_N, _HPAD), jnp.float32),
        ],
    )(x, W1, b1, W2p, b2p, adj)


_SC_INFO = plsc.get_sparse_core_info()
_NW = _SC_INFO.num_cores * _SC_INFO.num_subcores
_L = _SC_INFO.num_lanes
_EPW = _B // _NW
_EC = 128
_NEC = _EPW // _EC
_NG = _EC // _L


def _sc_score_body(table, head, pos, neg, pos_out, neg_out, l2_out,
                   hidx, pidx, nidx, bufs,
                   psc, nsc, l2st, sems):
    wid = lax.axis_index("s") * _SC_INFO.num_cores + lax.axis_index("c")
    e0 = wid * _EPW
    pltpu.sync_copy(head.at[pl.ds(e0, _EPW)], hidx)
    pltpu.sync_copy(pos.at[pl.ds(e0, _EPW)], pidx)
    pltpu.sync_copy(neg.at[pl.ds(e0, _EPW)], nidx)

    pending = []
    for c in range(_NEC):
        sl = pl.ds(c * _EC, _EC)
        hb, pb, nb = bufs[c]
        pending.append([
            pltpu.async_copy(table.at[hidx.at[sl]], hb, sems[c]),
            pltpu.async_copy(table.at[pidx.at[sl]], pb, sems[c]),
            pltpu.async_copy(table.at[nidx.at[sl]], nb, sems[c]),
        ])

    zero = jnp.zeros((_L,), jnp.float32)
    l2h = l2p = l2n = zero
    lane = lax.iota(jnp.int32, _L)
    for c in range(_NEC):
        for cp in pending[c]:
            cp.wait()
        hb, pb, nb = bufs[c]
        for g in range(_NG):
            rows = lane + (g * _L)

            ap = an = zero
            for d in range(_NHID):
                cd = (lane + d) & (_NHID - 1)
                hv = plsc.load_gather(hb, [rows, cd])
                pv = plsc.load_gather(pb, [rows, cd])
                nv = plsc.load_gather(nb, [rows, cd])
                ap = ap + hv * pv
                an = an + hv * nv
                l2h = l2h + hv * hv
                l2p = l2p + pv * pv
                l2n = l2n + nv * nv
            off = c * _EC + g * _L
            psc[pl.ds(off, _L)] = ap
            nsc[pl.ds(off, _L)] = an

    l2st[...] = l2h + l2p + l2n
    pltpu.sync_copy(psc, pos_out.at[pl.ds(e0, _EPW)])
    pltpu.sync_copy(nsc, neg_out.at[pl.ds(e0, _EPW)])
    pltpu.sync_copy(l2st, l2_out.at[pl.ds(wid * _L, _L)])


@functools.partial(
    pl.kernel,
    mesh=plsc.VectorSubcoreMesh(core_axis_name="c", subcore_axis_name="s"),
    out_type=[
        jax.ShapeDtypeStruct((_B,), jnp.float32),
        jax.ShapeDtypeStruct((_B,), jnp.float32),
        jax.ShapeDtypeStruct((_NW * _L,), jnp.float32),
    ],
    scratch_types=[
        pltpu.VMEM((_EPW,), jnp.int32),
        pltpu.VMEM((_EPW,), jnp.int32),
        pltpu.VMEM((_EPW,), jnp.int32),
        *[pltpu.VMEM((_EC, _NHID), jnp.float32) for _ in range(3 * _NEC)],
        pltpu.VMEM((_EPW,), jnp.float32),
        pltpu.VMEM((_EPW,), jnp.float32),
        pltpu.VMEM((_L,), jnp.float32),
        pltpu.SemaphoreType.DMA,
        pltpu.SemaphoreType.DMA,
        pltpu.SemaphoreType.DMA,
        pltpu.SemaphoreType.DMA,
    ],
    compiler_params=pltpu.CompilerParams(needs_layout_passes=False,
                                         use_tc_tiling_on_sc=False),
)
def _sc_score(table, head, pos, neg, pos_out, neg_out, l2_out,
              hidx, pidx, nidx, *rest):
    bufs = [tuple(rest[3 * c:3 * c + 3]) for c in range(_NEC)]
    psc, nsc, l2st = rest[3 * _NEC:3 * _NEC + 3]
    sems = list(rest[3 * _NEC + 3:])
    _sc_score_body(table, head, pos, neg, pos_out, neg_out, l2_out,
                   hidx, pidx, nidx, bufs, psc, nsc, l2st, sems)


def _loss_body(ps_ref, ns_ref, l2_ref, out_ref):
    z = ps_ref[...] - ns_ref[...]
    sp = jnp.maximum(-z, 0.0) + jnp.log(1.0 + jnp.exp(-jnp.abs(z)))
    out_ref[0, 0] = (jnp.sum(sp) * (1.0 / _B)
                     + (1e-5 * 0.5 / _B) * jnp.sum(l2_ref[...]))


def _loss_tc(ps2, ns2, l2r):
    out = pl.pallas_call(
        _loss_body,
        in_specs=[
            pl.BlockSpec((_B // 128, 128), lambda: (0, 0)),
            pl.BlockSpec((_B // 128, 128), lambda: (0, 0)),
            pl.BlockSpec((_NW * _L // 128, 128), lambda: (0, 0)),
        ],
        out_specs=pl.BlockSpec(memory_space=pltpu.SMEM),
        out_shape=jax.ShapeDtypeStruct((1, 1), jnp.float32),
    )(ps2, ns2, l2r)
    return out[0, 0]


def kernel(x, adj, head, pos_tail, neg_tail, W1, b1, W2, b2):
    pack = _HPAD // _NHID
    head = pack * head.astype(jnp.int32)
    pos_tail = pack * pos_tail.astype(jnp.int32)
    neg_tail = pack * neg_tail.astype(jnp.int32)
    W2p = jnp.pad(W2, ((0, 0), (0, _HPAD - _NHID)))
    b2p = jnp.pad(b2, (0, _HPAD - _NHID)).reshape(1, _HPAD)
    h = _gcn_tc(x, adj, W1, b1.reshape(1, _NHID), W2p, b2p)
    table = h.reshape(_N * pack, _NHID)
    ps, ns, l2 = _sc_score(table, head, pos_tail, neg_tail)
    return _loss_tc(ps.reshape(_B // 128, 128),
                    ns.reshape(_B // 128, 128),
                    l2.reshape(_NW * _L // 128, 128))


# speedup vs baseline: 1.0237x; 1.0237x over previous
import functools

import jax
import jax.numpy as jnp
from jax import lax
from jax.experimental import pallas as pl
from jax.experimental.pallas import tpu as pltpu
from jax.experimental.pallas import tpu_sc as plsc

_N = 10000
_NFEAT = 128
_NHID = 32
_HPAD = 128
_B = 16384

_BM = 400
_NBLK = _N // _BM


def _gcn_tc_body(x_ref, w1_ref, b1_ref, w2_ref, b2_ref, adj_ref, h_ref,
                 s1_scr, s2_scr):
    p = pl.program_id(0)
    i = pl.program_id(1)

    @pl.when((p == 0) & (i == 0))
    def _():
        s1_scr[...] = jnp.dot(x_ref[...], w1_ref[...],
                              preferred_element_type=jnp.float32)

    @pl.when(p == 0)
    def _():
        h1 = jnp.dot(adj_ref[...], s1_scr[...],
                     preferred_element_type=jnp.float32) + b1_ref[...]
        h1 = jnp.maximum(h1, 0.0)
        s2_scr[pl.ds(i * _BM, _BM), :] = jnp.dot(
            h1, w2_ref[...], preferred_element_type=jnp.float32)

    @pl.when(p == 1)
    def _():
        blk = _NBLK - 1 - i
        h_ref[pl.ds(blk * _BM, _BM), :] = jnp.dot(
            adj_ref[...], s2_scr[...],
            preferred_element_type=jnp.float32) + b2_ref[...]


def _gcn_tc(x, adj, W1, b1, W2p, b2p):
    return pl.pallas_call(
        _gcn_tc_body,
        grid=(2, _NBLK),
        in_specs=[
            pl.BlockSpec((_N, _NFEAT), lambda p, i: (0, 0)),
            pl.BlockSpec((_NFEAT, _NHID), lambda p, i: (0, 0)),
            pl.BlockSpec((1, _NHID), lambda p, i: (0, 0)),
            pl.BlockSpec((_NHID, _HPAD), lambda p, i: (0, 0)),
            pl.BlockSpec((1, _HPAD), lambda p, i: (0, 0)),
            pl.BlockSpec((_BM, _N),
                         lambda p, i: (jnp.where(p == 1, _NBLK - 1 - i, i),
                                       0)),
        ],
        out_specs=pl.BlockSpec((_N, _HPAD), lambda p, i: (0, 0)),
        out_shape=jax.ShapeDtypeStruct((_N, _HPAD), jnp.float32),
        scratch_shapes=[
            pltpu.VMEM((_N, _NHID), jnp.float32),
            pltpu.VMEM((_N, _HPAD), jnp.float32),
        ],
    )(x, W1, b1, W2p, b2p, adj)


_SC_INFO = plsc.get_sparse_core_info()
_NW = _SC_INFO.num_cores * _SC_INFO.num_subcores
_L = _SC_INFO.num_lanes
_EPW = _B // _NW
_EC = 128
_NEC = _EPW // _EC
_NG = _EC // _L


def _sc_score_body(table, head, pos, neg, pos_out, neg_out, l2_out,
                   hidx, pidx, nidx, bufs,
                   psc, nsc, l2st, sems):
    wid = lax.axis_index("s") * _SC_INFO.num_cores + lax.axis_index("c")
    e0 = wid * _EPW
    pltpu.sync_copy(head.at[pl.ds(e0, _EPW)], hidx)
    pltpu.sync_copy(pos.at[pl.ds(e0, _EPW)], pidx)
    pltpu.sync_copy(neg.at[pl.ds(e0, _EPW)], nidx)

    pending = []
    for c in range(_NEC):
        sl = pl.ds(c * _EC, _EC)
        hb, pb, nb = bufs[c]
        pending.append([
            pltpu.async_copy(table.at[hidx.at[sl]], hb, sems[c]),
            pltpu.async_copy(table.at[pidx.at[sl]], pb, sems[c]),
            pltpu.async_copy(table.at[nidx.at[sl]], nb, sems[c]),
        ])

    zero = jnp.zeros((_L,), jnp.float32)
    l2h = l2p = l2n = zero
    lane = lax.iota(jnp.int32, _L)
    for c in range(_NEC):
        for cp in pending[c]:
            cp.wait()
        hb, pb, nb = bufs[c]
        for g in range(_NG):
            rows = lane + (g * _L)

            ap = an = zero
            for d in range(_NHID):
                cd = (lane + d) & (_NHID - 1)
                hv = plsc.load_gather(hb, [rows, cd])
                pv = plsc.load_gather(pb, [rows, cd])
                nv = plsc.load_gather(nb, [rows, cd])
                ap = ap + hv * pv
                an = an + hv * nv
                l2h = l2h + hv * hv
                l2p = l2p + pv * pv
                l2n = l2n + nv * nv
            off = c * _EC + g * _L
            psc[pl.ds(off, _L)] = ap
            nsc[pl.ds(off, _L)] = an

    l2st[...] = l2h + l2p + l2n
    pltpu.sync_copy(psc, pos_out.at[pl.ds(e0, _EPW)])
    pltpu.sync_copy(nsc, neg_out.at[pl.ds(e0, _EPW)])
    pltpu.sync_copy(l2st, l2_out.at[pl.ds(wid * _L, _L)])


@functools.partial(
    pl.kernel,
    mesh=plsc.VectorSubcoreMesh(core_axis_name="c", subcore_axis_name="s"),
    out_type=[
        jax.ShapeDtypeStruct((_B,), jnp.float32),
        jax.ShapeDtypeStruct((_B,), jnp.float32),
        jax.ShapeDtypeStruct((_NW * _L,), jnp.float32),
    ],
    scratch_types=[
        pltpu.VMEM((_EPW,), jnp.int32),
        pltpu.VMEM((_EPW,), jnp.int32),
        pltpu.VMEM((_EPW,), jnp.int32),
        *[pltpu.VMEM((_EC, _NHID), jnp.float32) for _ in range(3 * _NEC)],
        pltpu.VMEM((_EPW,), jnp.float32),
        pltpu.VMEM((_EPW,), jnp.float32),
        pltpu.VMEM((_L,), jnp.float32),
        pltpu.SemaphoreType.DMA,
        pltpu.SemaphoreType.DMA,
        pltpu.SemaphoreType.DMA,
        pltpu.SemaphoreType.DMA,
    ],
    compiler_params=pltpu.CompilerParams(needs_layout_passes=False,
                                         use_tc_tiling_on_sc=False),
)
def _sc_score(table, head, pos, neg, pos_out, neg_out, l2_out,
              hidx, pidx, nidx, *rest):
    bufs = [tuple(rest[3 * c:3 * c + 3]) for c in range(_NEC)]
    psc, nsc, l2st = rest[3 * _NEC:3 * _NEC + 3]
    sems = list(rest[3 * _NEC + 3:])
    _sc_score_body(table, head, pos, neg, pos_out, neg_out, l2_out,
                   hidx, pidx, nidx, bufs, psc, nsc, l2st, sems)


def _loss_body(ps_ref, ns_ref, l2_ref, out_ref):
    z = ps_ref[...] - ns_ref[...]
    sp = jnp.maximum(-z, 0.0) + jnp.log(1.0 + jnp.exp(-jnp.abs(z)))
    out_ref[0, 0] = (jnp.sum(sp) * (1.0 / _B)
                     + (1e-5 * 0.5 / _B) * jnp.sum(l2_ref[...]))


def _loss_tc(ps2, ns2, l2r):
    out = pl.pallas_call(
        _loss_body,
        in_specs=[
            pl.BlockSpec((_B // 128, 128), lambda: (0, 0)),
            pl.BlockSpec((_B // 128, 128), lambda: (0, 0)),
            pl.BlockSpec((_NW * _L // 128, 128), lambda: (0, 0)),
        ],
        out_specs=pl.BlockSpec(memory_space=pltpu.SMEM),
        out_shape=jax.ShapeDtypeStruct((1, 1), jnp.float32),
    )(ps2, ns2, l2r)
    return out[0, 0]


def kernel(x, adj, head, pos_tail, neg_tail, W1, b1, W2, b2):
    pack = _HPAD // _NHID
    head = pack * head.astype(jnp.int32)
    pos_tail = pack * pos_tail.astype(jnp.int32)
    neg_tail = pack * neg_tail.astype(jnp.int32)
    W2p = jnp.pad(W2, ((0, 0), (0, _HPAD - _NHID)))
    b2p = jnp.pad(b2, (0, _HPAD - _NHID)).reshape(1, _HPAD)
    h = _gcn_tc(x, adj, W1, b1.reshape(1, _NHID), W2p, b2p)
    table = h.reshape(_N * pack, _NHID)
    ps, ns, l2 = _sc_score(table, head, pos_tail, neg_tail)
    return _loss_tc(ps.reshape(_B // 128, 128),
                    ns.reshape(_B // 128, 128),
                    l2.reshape(_NW * _L // 128, 128))


# speedup vs baseline: 1.0275x; 1.0038x over previous
import functools

import jax
import jax.numpy as jnp
from jax import lax
from jax.experimental import pallas as pl
from jax.experimental.pallas import tpu as pltpu
from jax.experimental.pallas import tpu_sc as plsc

_N = 10000
_NFEAT = 128
_NHID = 32
_HPAD = 128
_B = 16384

_BM = 400
_NBLK = _N // _BM


def _gcn_tc_body(x_ref, w1_ref, b1_ref, w2_ref, b2_ref, adj_ref, h_ref,
                 s1_scr, s2_scr):
    p = pl.program_id(0)
    i = pl.program_id(1)

    @pl.when((p == 0) & (i == 0))
    def _():
        s1_scr[...] = jnp.dot(x_ref[...], w1_ref[...],
                              preferred_element_type=jnp.float32)

    @pl.when(p == 0)
    def _():
        h1 = jnp.dot(adj_ref[...], s1_scr[...],
                     preferred_element_type=jnp.float32) + b1_ref[...]
        h1 = jnp.maximum(h1, 0.0)
        s2_scr[pl.ds(i * _BM, _BM), :] = jnp.dot(
            h1, w2_ref[...], preferred_element_type=jnp.float32)

    @pl.when(p == 1)
    def _():
        blk = _NBLK - 1 - i
        h_ref[pl.ds(blk * _BM, _BM), :] = jnp.dot(
            adj_ref[...], s2_scr[...],
            preferred_element_type=jnp.float32) + b2_ref[...]


def _gcn_tc(x, adj, W1, b1, W2p, b2p):
    return pl.pallas_call(
        _gcn_tc_body,
        grid=(2, _NBLK),
        in_specs=[
            pl.BlockSpec((_N, _NFEAT), lambda p, i: (0, 0)),
            pl.BlockSpec((_NFEAT, _NHID), lambda p, i: (0, 0)),
            pl.BlockSpec((1, _NHID), lambda p, i: (0, 0)),
            pl.BlockSpec((_NHID, _HPAD), lambda p, i: (0, 0)),
            pl.BlockSpec((1, _HPAD), lambda p, i: (0, 0)),
            pl.BlockSpec((_BM, _N),
                         lambda p, i: (jnp.where(p == 1, _NBLK - 1 - i, i),
                                       0)),
        ],
        out_specs=pl.BlockSpec((_N, _HPAD), lambda p, i: (0, 0)),
        out_shape=jax.ShapeDtypeStruct((_N, _HPAD), jnp.float32),
        scratch_shapes=[
            pltpu.VMEM((_N, _NHID), jnp.float32),
            pltpu.VMEM((_N, _HPAD), jnp.float32),
        ],
    )(x, W1, b1, W2p, b2p, adj)


_SC_INFO = plsc.get_sparse_core_info()
_NW = _SC_INFO.num_cores * _SC_INFO.num_subcores
_L = _SC_INFO.num_lanes
_EPW = _B // _NW
_EC = 128
_NEC = _EPW // _EC
_NG = _EC // _L


def _sc_score_body(table, head, pos, neg, pos_out, neg_out, l2_out,
                   hidx, pidx, nidx, bufs,
                   psc, nsc, l2st, sems):
    wid = lax.axis_index("s") * _SC_INFO.num_cores + lax.axis_index("c")
    e0 = wid * _EPW
    pltpu.sync_copy(head.at[pl.ds(e0, _EPW)], hidx)
    pltpu.sync_copy(pos.at[pl.ds(e0, _EPW)], pidx)
    pltpu.sync_copy(neg.at[pl.ds(e0, _EPW)], nidx)

    def issue(c):
        hb, pb, nb = bufs[c % 2]
        sem = sems[c % 2]
        sl = pl.ds(c * _EC, _EC)
        return [
            pltpu.async_copy(table.at[hidx.at[sl]], hb, sem),
            pltpu.async_copy(table.at[pidx.at[sl]], pb, sem),
            pltpu.async_copy(table.at[nidx.at[sl]], nb, sem),
        ]

    zero = jnp.zeros((_L,), jnp.float32)
    l2h = l2p = l2n = zero
    lane = lax.iota(jnp.int32, _L)
    pending = issue(0)
    for c in range(_NEC):
        nxt = issue(c + 1) if c + 1 < _NEC else []
        for cp in pending:
            cp.wait()
        hb, pb, nb = bufs[c % 2]
        for g in range(_NG):
            rows = lane + (g * _L)

            ap = an = zero
            for d in range(_NHID):
                cd = (lane + d) & (_NHID - 1)
                hv = plsc.load_gather(hb, [rows, cd])
                pv = plsc.load_gather(pb, [rows, cd])
                nv = plsc.load_gather(nb, [rows, cd])
                ap = ap + hv * pv
                an = an + hv * nv
                l2h = l2h + hv * hv
                l2p = l2p + pv * pv
                l2n = l2n + nv * nv
            off = c * _EC + g * _L
            psc[pl.ds(off, _L)] = ap
            nsc[pl.ds(off, _L)] = an
        pending = nxt

    l2st[...] = l2h + l2p + l2n
    pltpu.sync_copy(psc, pos_out.at[pl.ds(e0, _EPW)])
    pltpu.sync_copy(nsc, neg_out.at[pl.ds(e0, _EPW)])
    pltpu.sync_copy(l2st, l2_out.at[pl.ds(wid * _L, _L)])


@functools.partial(
    pl.kernel,
    mesh=plsc.VectorSubcoreMesh(core_axis_name="c", subcore_axis_name="s"),
    out_type=[
        jax.ShapeDtypeStruct((_B,), jnp.float32),
        jax.ShapeDtypeStruct((_B,), jnp.float32),
        jax.ShapeDtypeStruct((_NW * _L,), jnp.float32),
    ],
    scratch_types=[
        pltpu.VMEM((_EPW,), jnp.int32),
        pltpu.VMEM((_EPW,), jnp.int32),
        pltpu.VMEM((_EPW,), jnp.int32),
        *[pltpu.VMEM((_EC, _NHID), jnp.float32) for _ in range(6)],
        pltpu.VMEM((_EPW,), jnp.float32),
        pltpu.VMEM((_EPW,), jnp.float32),
        pltpu.VMEM((_L,), jnp.float32),
        pltpu.SemaphoreType.DMA,
        pltpu.SemaphoreType.DMA,
    ],
    compiler_params=pltpu.CompilerParams(needs_layout_passes=False,
                                         use_tc_tiling_on_sc=False),
)
def _sc_score(table, head, pos, neg, pos_out, neg_out, l2_out,
              hidx, pidx, nidx, *rest):
    bufs = [tuple(rest[0:3]), tuple(rest[3:6])]
    psc, nsc, l2st = rest[6:9]
    sems = list(rest[9:])
    _sc_score_body(table, head, pos, neg, pos_out, neg_out, l2_out,
                   hidx, pidx, nidx, bufs, psc, nsc, l2st, sems)


def _loss_body(ps_ref, ns_ref, l2_ref, out_ref):
    z = ps_ref[...] - ns_ref[...]
    sp = jnp.maximum(-z, 0.0) + jnp.log(1.0 + jnp.exp(-jnp.abs(z)))
    out_ref[0, 0] = (jnp.sum(sp) * (1.0 / _B)
                     + (1e-5 * 0.5 / _B) * jnp.sum(l2_ref[...]))


def _loss_tc(ps2, ns2, l2r):
    out = pl.pallas_call(
        _loss_body,
        in_specs=[
            pl.BlockSpec((_B // 128, 128), lambda: (0, 0)),
            pl.BlockSpec((_B // 128, 128), lambda: (0, 0)),
            pl.BlockSpec((_NW * _L // 128, 128), lambda: (0, 0)),
        ],
        out_specs=pl.BlockSpec(memory_space=pltpu.SMEM),
        out_shape=jax.ShapeDtypeStruct((1, 1), jnp.float32),
    )(ps2, ns2, l2r)
    return out[0, 0]


def kernel(x, adj, head, pos_tail, neg_tail, W1, b1, W2, b2):
    pack = _HPAD // _NHID
    head = pack * head.astype(jnp.int32)
    pos_tail = pack * pos_tail.astype(jnp.int32)
    neg_tail = pack * neg_tail.astype(jnp.int32)
    W2p = jnp.pad(W2, ((0, 0), (0, _HPAD - _NHID)))
    b2p = jnp.pad(b2, (0, _HPAD - _NHID)).reshape(1, _HPAD)
    h = _gcn_tc(x, adj, W1, b1.reshape(1, _NHID), W2p, b2p)
    table = h.reshape(_N * pack, _NHID)
    ps, ns, l2 = _sc_score(table, head, pos_tail, neg_tail)
    return _loss_tc(ps.reshape(_B // 128, 128),
                    ns.reshape(_B // 128, 128),
                    l2.reshape(_NW * _L // 128, 128))
